# baseline (speedup 1.0000x reference)
import functools

import jax
import jax.numpy as jnp
from jax import lax
from jax.experimental import pallas as pl
from jax.experimental.pallas import tpu as pltpu
from jax.experimental.pallas import tpu_sc as plsc

B = 4
T = 2048
H = 1024
EPS = 1e-12

_NC = 2
_NS = 16
_NW = _NC * _NS
_PPW = T // _NW
_P = 16
_NCHUNK = _PPW // _P
_CTOK = B * _P


def _rsqrt(v):
    i = plsc.bitcast(v, jnp.int32)
    y = plsc.bitcast(jnp.int32(0x5F3759DF) - (i >> 1), jnp.float32)
    for _ in range(4):
        y = y * (1.5 - 0.5 * v * y * y)
    return y


def _body(ids_hbm, tt_hbm, word_hbm, pos_hbm, type_hbm, gamma_hbm, beta_hbm,
          out_hbm, idx_all, tt_all, xbuf, pos_buf, type_buf, gamma_buf,
          beta_buf, sem):
    w = lax.axis_index("s") * _NC + lax.axis_index("c")
    base_p = w * _PPW

    for b in range(B):
        pltpu.sync_copy(ids_hbm.at[pl.ds(b * T + base_p, _PPW)],
                        idx_all.at[pl.ds(b * _PPW, _PPW)])
        pltpu.sync_copy(tt_hbm.at[pl.ds(b * T + base_p, _PPW)],
                        tt_all.at[pl.ds(b * _PPW, _PPW)])
    pltpu.sync_copy(type_hbm, type_buf)
    pltpu.sync_copy(gamma_hbm, gamma_buf)
    pltpu.sync_copy(beta_hbm, beta_buf)

    lane = lax.iota(jnp.int32, 16)

    for c in range(_NCHUNK):
        pltpu.sync_copy(pos_hbm.at[pl.ds(base_p + c * _P, _P)], pos_buf)

        cps = []
        for g in range(B):
            idx16 = idx_all[pl.ds(g * _PPW + c * _P, _P)]
            cps.append(pltpu.async_copy(word_hbm.at[idx16],
                                        xbuf.at[pl.ds(g * 16, 16)], sem))
        for cp in cps:
            cp.wait()

        for g in range(B):
            rows = lane + g * 16
            tt16 = tt_all[pl.ds(g * _PPW + c * _P, _P)]

            def p1(j, carry):
                acc, acc2 = carry
                col = jnp.full((16,), 0, jnp.int32) + j
                x = (plsc.load_gather(xbuf, [rows, col])
                     + plsc.load_gather(pos_buf, [lane, col])
                     + plsc.load_gather(type_buf, [tt16, col]))
                plsc.store_scatter(xbuf, [rows, col], x)
                return acc + x, acc2 + x * x

            zeros = jnp.zeros((16,), jnp.float32)
            acc, acc2 = lax.fori_loop(0, H, p1, (zeros, zeros))
            mean = acc * (1.0 / H)
            var = acc2 * (1.0 / H) - mean * mean
            rstd = _rsqrt(var + EPS)

            def p2(j, _):
                col = jnp.full((16,), 0, jnp.int32) + j
                gs = plsc.load_gather(gamma_buf, [col])
                bs = plsc.load_gather(beta_buf, [col])
                x = plsc.load_gather(xbuf, [rows, col])
                y = (x - mean) * rstd * gs + bs
                plsc.store_scatter(xbuf, [rows, col], y)
                return 0

            lax.fori_loop(0, H, p2, 0)

        for b in range(B):
            pltpu.sync_copy(xbuf.at[pl.ds(b * 16, 16)],
                            out_hbm.at[pl.ds(b * T + base_p + c * _P, _P)])


@jax.jit
def _run(ids_flat, tt_flat, word_emb, pos_emb, type_emb, ln_gamma, ln_beta):
    mesh = plsc.VectorSubcoreMesh(core_axis_name="c", subcore_axis_name="s")
    f = functools.partial(
        pl.kernel,
        mesh=mesh,
        compiler_params=pltpu.CompilerParams(
            use_tc_tiling_on_sc=False, needs_layout_passes=False),
        out_type=jax.ShapeDtypeStruct((B * T, H), jnp.float32),
        scratch_types=[
            pltpu.VMEM((B * _PPW,), jnp.int32),
            pltpu.VMEM((B * _PPW,), jnp.int32),
            pltpu.VMEM((_CTOK, H), jnp.float32),
            pltpu.VMEM((_P, H), jnp.float32),
            pltpu.VMEM((2, H), jnp.float32),
            pltpu.VMEM((H,), jnp.float32),
            pltpu.VMEM((H,), jnp.float32),
            pltpu.SemaphoreType.DMA,
        ],
    )(_body)
    return f(ids_flat, tt_flat, word_emb, pos_emb, type_emb, ln_gamma, ln_beta)


def kernel(input_ids, token_type_ids, word_emb, pos_emb, type_emb, ln_gamma,
           ln_beta):
    assert input_ids.shape == (B, T) and word_emb.shape[1] == H
    ids_flat = input_ids.reshape(-1).astype(jnp.int32)
    tt_flat = token_type_ids.reshape(-1).astype(jnp.int32)
    out = _run(ids_flat, tt_flat, word_emb, pos_emb, type_emb, ln_gamma,
               ln_beta)
    return out.reshape(B, T, H)


# speedup vs baseline: 2.0672x; 2.0672x over previous
import functools

import jax
import jax.numpy as jnp
from jax import lax
from jax.experimental import pallas as pl
from jax.experimental.pallas import tpu as pltpu
from jax.experimental.pallas import tpu_sc as plsc

B = 4
T = 2048
H = 1024
EPS = 1e-12

_NC = 2
_NS = 16
_NW = _NC * _NS
_PPW = T // _NW
_P = 16
_NCHUNK = _PPW // _P
_CTOK = B * _P


def _rsqrt(v):
    i = plsc.bitcast(v, jnp.int32)
    y = plsc.bitcast(jnp.int32(0x5F3759DF) - (i >> 1), jnp.float32)
    for _ in range(4):
        y = y * (1.5 - 0.5 * v * y * y)
    return y


def _body(ids_hbm, tt_hbm, word_hbm, pos_hbm, type_hbm, gamma_hbm, beta_hbm,
          out_hbm, idx_all, tt_all, xbuf, pos_buf, type_buf, gamma_buf,
          beta_buf, gsem, osem):
    w = lax.axis_index("s") * _NC + lax.axis_index("c")
    base_p = w * _PPW

    for b in range(B):
        pltpu.sync_copy(ids_hbm.at[pl.ds(b * T + base_p, _PPW)],
                        idx_all.at[pl.ds(b * _PPW, _PPW)])
        pltpu.sync_copy(tt_hbm.at[pl.ds(b * T + base_p, _PPW)],
                        tt_all.at[pl.ds(b * _PPW, _PPW)])
    pltpu.sync_copy(type_hbm, type_buf)
    pltpu.sync_copy(gamma_hbm, gamma_buf)
    pltpu.sync_copy(beta_hbm, beta_buf)

    lane = lax.iota(jnp.int32, 16)
    zeros_i = jnp.zeros((16,), jnp.int32)
    ones_i = jnp.ones((16,), jnp.int32)
    rows_l = [lane + g * 16 for g in range(B)]
    zf = jnp.zeros((16,), jnp.float32)

    out_cps = []
    for c in range(_NCHUNK):
        pltpu.sync_copy(pos_hbm.at[pl.ds(base_p + c * _P, _P)], pos_buf)

        for cp in out_cps:
            cp.wait()
        out_cps = []

        cps = []
        for g in range(B):
            idx16 = idx_all[pl.ds(g * _PPW + c * _P, _P)]
            cps.append(pltpu.async_copy(word_hbm.at[idx16],
                                        xbuf.at[pl.ds(g * 16, 16)], gsem))
        for cp in cps:
            cp.wait()

        tt16f = [tt_all[pl.ds(g * _PPW + c * _P, _P)].astype(jnp.float32)
                 for g in range(B)]

        init = (tuple(zf for _ in range(B)), tuple(zf for _ in range(B)))

        @plsc.parallel_loop(0, H, carry=init, unroll=4)
        def p1(j, carry):
            accs, acc2s = carry
            col = zeros_i + j
            posv = plsc.load_gather(pos_buf, [lane, col])
            t0 = plsc.load_gather(type_buf, [zeros_i, col])
            t1 = plsc.load_gather(type_buf, [ones_i, col])
            diff = t1 - t0
            base = posv + t0
            na, na2 = [], []
            for g in range(B):
                x = plsc.load_gather(xbuf, [rows_l[g], col])
                x = x + base + tt16f[g] * diff
                plsc.store_scatter(xbuf, [rows_l[g], col], x)
                na.append(accs[g] + x)
                na2.append(acc2s[g] + x * x)
            return tuple(na), tuple(na2)

        accs, acc2s = p1
        means = [a * (1.0 / H) for a in accs]
        rstds = [_rsqrt(a2 * (1.0 / H) - m * m + EPS)
                 for a2, m in zip(acc2s, means)]

        @plsc.parallel_loop(0, H, unroll=4)
        def p2(j):
            col = zeros_i + j
            gs = plsc.load_gather(gamma_buf, [col])
            bs = plsc.load_gather(beta_buf, [col])
            for g in range(B):
                x = plsc.load_gather(xbuf, [rows_l[g], col])
                y = (x - means[g]) * rstds[g] * gs + bs
                plsc.store_scatter(xbuf, [rows_l[g], col], y)

        for b in range(B):
            out_cps.append(pltpu.async_copy(
                xbuf.at[pl.ds(b * 16, 16)],
                out_hbm.at[pl.ds(b * T + base_p + c * _P, _P)], osem))
    for cp in out_cps:
        cp.wait()


@jax.jit
def _run(ids_flat, tt_flat, word_emb, pos_emb, type_emb, ln_gamma, ln_beta):
    mesh = plsc.VectorSubcoreMesh(core_axis_name="c", subcore_axis_name="s")
    f = functools.partial(
        pl.kernel,
        mesh=mesh,
        compiler_params=pltpu.CompilerParams(
            use_tc_tiling_on_sc=False, needs_layout_passes=False),
        out_type=jax.ShapeDtypeStruct((B * T, H), jnp.float32),
        scratch_types=[
            pltpu.VMEM((B * _PPW,), jnp.int32),
            pltpu.VMEM((B * _PPW,), jnp.int32),
            pltpu.VMEM((_CTOK, H), jnp.float32),
            pltpu.VMEM((_P, H), jnp.float32),
            pltpu.VMEM((2, H), jnp.float32),
            pltpu.VMEM((H,), jnp.float32),
            pltpu.VMEM((H,), jnp.float32),
            pltpu.SemaphoreType.DMA,
            pltpu.SemaphoreType.DMA,
        ],
    )(_body)
    return f(ids_flat, tt_flat, word_emb, pos_emb, type_emb, ln_gamma,
             ln_beta)


def kernel(input_ids, token_type_ids, word_emb, pos_emb, type_emb, ln_gamma,
           ln_beta):
    assert input_ids.shape == (B, T) and word_emb.shape[1] == H
    ids_flat = input_ids.reshape(-1).astype(jnp.int32)
    tt_flat = token_type_ids.reshape(-1).astype(jnp.int32)
    out = _run(ids_flat, tt_flat, word_emb, pos_emb, type_emb, ln_gamma,
               ln_beta)
    return out.reshape(B, T, H)


# speedup vs baseline: 3.4538x; 1.6708x over previous
import functools

import jax
import jax.numpy as jnp
from jax import lax
from jax.experimental import pallas as pl
from jax.experimental.pallas import tpu as pltpu
from jax.experimental.pallas import tpu_sc as plsc

B = 4
T = 2048
H = 1024
EPS = 1e-12

_NC = 2
_NS = 16
_NW = _NC * _NS
_PPW = T // _NW
_P = 16
_NH = 2 * (_PPW // _P)
_G = 2


def _rsqrt(v):
    i = plsc.bitcast(v, jnp.int32)
    y = plsc.bitcast(jnp.int32(0x5F3759DF) - (i >> 1), jnp.float32)
    for _ in range(4):
        y = y * (1.5 - 0.5 * v * y * y)
    return y


def _body(ids_hbm, tt_hbm, word_hbm, pos_hbm, type_hbm, gamma_hbm, beta_hbm,
          out_hbm, idx_all, tt_all, xbuf, gamma_buf, beta_buf, gsem, osem):
    w = lax.axis_index("s") * _NC + lax.axis_index("c")
    base_p = w * _PPW

    for b in range(B):
        pltpu.sync_copy(ids_hbm.at[pl.ds(b * T + base_p, _PPW)],
                        idx_all.at[pl.ds(b * _PPW, _PPW)])
        pltpu.sync_copy(tt_hbm.at[pl.ds(b * T + base_p, _PPW)],
                        tt_all.at[pl.ds(b * _PPW, _PPW)])
    pltpu.sync_copy(gamma_hbm, gamma_buf)
    pltpu.sync_copy(beta_hbm, beta_buf)

    lane = lax.iota(jnp.int32, 16)
    zeros_i = jnp.zeros((16,), jnp.int32)
    zf = jnp.zeros((16,), jnp.float32)

    def batches(h):
        return (2 * (h % 2), 2 * (h % 2) + 1)

    def region(h):
        return 32 * (h % 2)

    def stage_and_gather(h):
        p0 = base_p + (h // 2) * _P
        r = region(h)
        for i, b in enumerate(batches(h)):
            dst = xbuf.at[pl.ds(r + 16 * i, 16)]
            pltpu.sync_copy(pos_hbm.at[pl.ds(p0, _P)], dst)
        cps = []
        for i, b in enumerate(batches(h)):
            dst = xbuf.at[pl.ds(r + 16 * i, 16)]
            idxr = idx_all.at[pl.ds(b * _PPW + (h // 2) * _P, _P)]
            ttr = tt_all.at[pl.ds(b * _PPW + (h // 2) * _P, _P)]
            cps.append(pltpu.async_copy(word_hbm.at[idxr], dst, gsem,
                                        add=True))
            cps.append(pltpu.async_copy(type_hbm.at[ttr], dst, gsem,
                                        add=True))
        return cps

    def writeback(h):
        p0 = base_p + (h // 2) * _P
        r = region(h)
        cps = []
        for i, b in enumerate(batches(h)):
            cps.append(pltpu.async_copy(
                xbuf.at[pl.ds(r + 16 * i, 16)],
                out_hbm.at[pl.ds(b * T + p0, _P)], osem))
        return cps

    def compute(h):
        rows_l = [lane + region(h) + 16 * i for i in range(_G)]
        init = (tuple(zf for _ in range(_G)), tuple(zf for _ in range(_G)))


        @plsc.parallel_loop(0, H, carry=init, unroll=4)
        def p1(j, carry):
            accs, acc2s = carry
            dcol = (lane + j) & (H - 1)
            na, na2 = [], []
            for i in range(_G):
                x = plsc.load_gather(xbuf, [rows_l[i], dcol])
                na.append(accs[i] + x)
                na2.append(acc2s[i] + x * x)
            return tuple(na), tuple(na2)

        accs, acc2s = p1
        means = [a * (1.0 / H) for a in accs]
        rstds = [_rsqrt(a2 * (1.0 / H) - m * m + EPS)
                 for a2, m in zip(acc2s, means)]

        @plsc.parallel_loop(0, H, unroll=4)
        def p2(j):
            dcol = (lane + j) & (H - 1)
            gs = plsc.load_gather(gamma_buf, [dcol])
            bs = plsc.load_gather(beta_buf, [dcol])
            for i in range(_G):
                x = plsc.load_gather(xbuf, [rows_l[i], dcol])
                y = (x - means[i]) * rstds[i] * gs + bs
                plsc.store_scatter(xbuf, [rows_l[i], dcol], y)

    gath = {0: stage_and_gather(0)}
    outs = {}
    for h in range(_NH):
        if h + 1 < _NH:
            for cp in outs.pop(h - 1, []):
                cp.wait()
            gath[h + 1] = stage_and_gather(h + 1)
        for cp in gath.pop(h):
            cp.wait()
        compute(h)
        outs[h] = writeback(h)
    for cps in outs.values():
        for cp in cps:
            cp.wait()


@jax.jit
def _run(ids_flat, tt_flat, word_emb, pos_emb, type_emb, ln_gamma, ln_beta):
    mesh = plsc.VectorSubcoreMesh(core_axis_name="c", subcore_axis_name="s")
    f = functools.partial(
        pl.kernel,
        mesh=mesh,
        compiler_params=pltpu.CompilerParams(
            use_tc_tiling_on_sc=False, needs_layout_passes=False),
        out_type=jax.ShapeDtypeStruct((B * T, H), jnp.float32),
        scratch_types=[
            pltpu.VMEM((B * _PPW,), jnp.int32),
            pltpu.VMEM((B * _PPW,), jnp.int32),
            pltpu.VMEM((64, H), jnp.float32),
            pltpu.VMEM((H,), jnp.float32),
            pltpu.VMEM((H,), jnp.float32),
            pltpu.SemaphoreType.DMA,
            pltpu.SemaphoreType.DMA,
        ],
    )(_body)
    return f(ids_flat, tt_flat, word_emb, pos_emb, type_emb, ln_gamma,
             ln_beta)


def kernel(input_ids, token_type_ids, word_emb, pos_emb, type_emb, ln_gamma,
           ln_beta):
    assert input_ids.shape == (B, T) and word_emb.shape[1] == H
    ids_flat = input_ids.reshape(-1).astype(jnp.int32)
    tt_flat = token_type_ids.reshape(-1).astype(jnp.int32)
    out = _run(ids_flat, tt_flat, word_emb, pos_emb, type_emb, ln_gamma,
               ln_beta)
    return out.reshape(B, T, H)


# speedup vs baseline: 3.5194x; 1.0190x over previous
import functools

import jax
import jax.numpy as jnp
from jax import lax
from jax.experimental import pallas as pl
from jax.experimental.pallas import tpu as pltpu
from jax.experimental.pallas import tpu_sc as plsc

B = 4
T = 2048
H = 1024
EPS = 1e-12

_NC = 2
_NS = 16
_NW = _NC * _NS
_PPW = T // _NW
_P = 16
_NH = 2 * (_PPW // _P)
_G = 2


def _rsqrt(v):
    i = plsc.bitcast(v, jnp.int32)
    y = plsc.bitcast(jnp.int32(0x5F3759DF) - (i >> 1), jnp.float32)
    for _ in range(4):
        y = y * (1.5 - 0.5 * v * y * y)
    return y


def _body(ids_hbm, tt_hbm, word_hbm, pos_hbm, type_hbm, gamma_hbm, beta_hbm,
          out_hbm, idx_all, tt_all, xbuf, gamma_buf, beta_buf, gsem, osem):
    w = lax.axis_index("s") * _NC + lax.axis_index("c")
    base_p = w * _PPW

    for b in range(B):
        pltpu.sync_copy(ids_hbm.at[b, pl.ds(base_p, _PPW)],
                        idx_all.at[pl.ds(b * _PPW, _PPW)])
        pltpu.sync_copy(tt_hbm.at[b, pl.ds(base_p, _PPW)],
                        tt_all.at[pl.ds(b * _PPW, _PPW)])
    pltpu.sync_copy(gamma_hbm, gamma_buf)
    pltpu.sync_copy(beta_hbm, beta_buf)

    lane = lax.iota(jnp.int32, 16)
    zeros_i = jnp.zeros((16,), jnp.int32)
    zf = jnp.zeros((16,), jnp.float32)

    def batches(h):
        return (2 * (h % 2), 2 * (h % 2) + 1)

    def region(h):
        return 32 * (h % 2)

    def stage_and_gather(h):
        p0 = base_p + (h // 2) * _P
        r = region(h)
        for i, b in enumerate(batches(h)):
            dst = xbuf.at[pl.ds(r + 16 * i, 16)]
            pltpu.sync_copy(pos_hbm.at[pl.ds(p0, _P)], dst)
        cps = []
        for i, b in enumerate(batches(h)):
            dst = xbuf.at[pl.ds(r + 16 * i, 16)]
            idxr = idx_all.at[pl.ds(b * _PPW + (h // 2) * _P, _P)]
            ttr = tt_all.at[pl.ds(b * _PPW + (h // 2) * _P, _P)]
            cps.append(pltpu.async_copy(word_hbm.at[idxr], dst, gsem,
                                        add=True))
            cps.append(pltpu.async_copy(type_hbm.at[ttr], dst, gsem,
                                        add=True))
        return cps

    def writeback(h):
        p0 = base_p + (h // 2) * _P
        r = region(h)
        cps = []
        for i, b in enumerate(batches(h)):
            cps.append(pltpu.async_copy(
                xbuf.at[pl.ds(r + 16 * i, 16)],
                out_hbm.at[pl.ds(b * T + p0, _P)], osem))
        return cps

    def compute(h):
        r = region(h)

        @plsc.parallel_loop(0, _P, unroll=1)
        def ptok(t):
            r0 = r + 2 * t

            @plsc.parallel_loop(0, H // 16, carry=(zf, zf, zf, zf),
                                unroll=8)
            def p1(jj, carry):
                a0, q0, a1, q1 = carry
                c = jj * 16
                x0 = xbuf[r0, pl.ds(c, 16)]
                x1 = xbuf[r0 + 1, pl.ds(c, 16)]
                return a0 + x0, q0 + x0 * x0, a1 + x1, q1 + x1 * x1

            a0, q0, a1, q1 = p1
            stats = []
            for a, q in ((a0, q0), (a1, q1)):
                mean = jnp.sum(a) * (1.0 / H)
                var = jnp.sum(q) * (1.0 / H) - mean * mean
                stats.append((zf + mean, _rsqrt(zf + var + EPS)))
            (m0, s0), (m1, s1) = stats

            @plsc.parallel_loop(0, H // 16, unroll=8)
            def p2(jj):
                c = jj * 16
                g = gamma_buf[pl.ds(c, 16)]
                bv = beta_buf[pl.ds(c, 16)]
                x0 = xbuf[r0, pl.ds(c, 16)]
                x1 = xbuf[r0 + 1, pl.ds(c, 16)]
                xbuf[r0, pl.ds(c, 16)] = (x0 - m0) * s0 * g + bv
                xbuf[r0 + 1, pl.ds(c, 16)] = (x1 - m1) * s1 * g + bv

    gath = {0: stage_and_gather(0)}
    outs = {}
    for h in range(_NH):
        if h + 1 < _NH:
            for cp in outs.pop(h - 1, []):
                cp.wait()
            gath[h + 1] = stage_and_gather(h + 1)
        for cp in gath.pop(h):
            cp.wait()
        compute(h)
        outs[h] = writeback(h)
    for cps in outs.values():
        for cp in cps:
            cp.wait()


@jax.jit
def _run(ids_flat, tt_flat, word_emb, pos_emb, type_emb, ln_gamma, ln_beta):
    mesh = plsc.VectorSubcoreMesh(core_axis_name="c", subcore_axis_name="s")
    f = functools.partial(
        pl.kernel,
        mesh=mesh,
        compiler_params=pltpu.CompilerParams(
            use_tc_tiling_on_sc=False, needs_layout_passes=False),
        out_type=jax.ShapeDtypeStruct((B * T, H), jnp.float32),
        scratch_types=[
            pltpu.VMEM((B * _PPW,), jnp.int32),
            pltpu.VMEM((B * _PPW,), jnp.int32),
            pltpu.VMEM((64, H), jnp.float32),
            pltpu.VMEM((H,), jnp.float32),
            pltpu.VMEM((H,), jnp.float32),
            pltpu.SemaphoreType.DMA,
            pltpu.SemaphoreType.DMA,
        ],
    )(_body)
    return f(ids_flat, tt_flat, word_emb, pos_emb, type_emb, ln_gamma,
             ln_beta)


def kernel(input_ids, token_type_ids, word_emb, pos_emb, type_emb, ln_gamma,
           ln_beta):
    assert input_ids.shape == (B, T) and word_emb.shape[1] == H
    out = _run(input_ids.astype(jnp.int32), token_type_ids.astype(jnp.int32),
               word_emb, pos_emb[:T], type_emb, ln_gamma, ln_beta)
    return out.reshape(B, T, H)


# speedup vs baseline: 7.4149x; 2.1069x over previous
import functools

import jax
import jax.numpy as jnp
from jax import lax
from jax.experimental import pallas as pl
from jax.experimental.pallas import tpu as pltpu
from jax.experimental.pallas import tpu_sc as plsc

B = 4
T = 2048
H = 1024
EPS = 1e-12

_NC = 2
_NS = 16
_NW = _NC * _NS
_RPW = (B * T) // _NW
_CH = 32
_NCHUNK = _RPW // _CH


def _gather_body(ids_hbm, word_hbm, out_hbm, idx_v, gbuf, gsem, osem):
    w = lax.axis_index("s") * _NC + lax.axis_index("c")
    base = w * _RPW
    pltpu.sync_copy(ids_hbm.at[pl.ds(base, _RPW)], idx_v)

    def gather(c):
        r = _CH * (c % 2)
        idxr = idx_v.at[pl.ds(c * _CH, _CH)]
        return [pltpu.async_copy(word_hbm.at[idxr],
                                 gbuf.at[pl.ds(r, _CH)], gsem)]

    def writeback(c):
        r = _CH * (c % 2)
        return [pltpu.async_copy(gbuf.at[pl.ds(r, _CH)],
                                 out_hbm.at[pl.ds(base + c * _CH, _CH)],
                                 osem)]

    gath = {0: gather(0)}
    outs = {}
    for c in range(_NCHUNK):
        for cp in gath.pop(c):
            cp.wait()
        if c + 1 < _NCHUNK:
            for cp in outs.pop(c - 1, []):
                cp.wait()
            gath[c + 1] = gather(c + 1)
        outs[c] = writeback(c)
    for cps in outs.values():
        for cp in cps:
            cp.wait()


def _ln_kernel(x_ref, pos_ref, tt_ref, type_ref, gamma_ref, beta_ref,
               out_ref):
    x = x_ref[...] + pos_ref[...]
    t0 = type_ref[0][None, :]
    t1 = type_ref[1][None, :]
    ttf = tt_ref[0, 0, :].astype(jnp.float32).reshape(-1, 1)
    x = x + t0 + ttf * (t1 - t0)
    mean = jnp.mean(x, axis=-1, keepdims=True)
    xc = x - mean
    var = jnp.mean(xc * xc, axis=-1, keepdims=True)
    normed = xc * jax.lax.rsqrt(var + EPS)
    out_ref[...] = normed * gamma_ref[0][None, :] + beta_ref[0][None, :]


@jax.jit
def _run(ids_flat, tt3d, word_emb, pos_emb, type_emb, gamma2d, beta2d):
    mesh = plsc.VectorSubcoreMesh(core_axis_name="c", subcore_axis_name="s")
    scratch = pl.kernel(
        _gather_body,
        mesh=mesh,
        compiler_params=pltpu.CompilerParams(
            use_tc_tiling_on_sc=False, needs_layout_passes=False),
        out_type=jax.ShapeDtypeStruct((B * T, H), jnp.float32),
        scratch_types=[
            pltpu.VMEM((_RPW,), jnp.int32),
            pltpu.VMEM((2 * _CH, H), jnp.float32),
            pltpu.SemaphoreType.DMA,
            pltpu.SemaphoreType.DMA,
        ],
    )(ids_flat, word_emb)

    blk = 128
    nblk = (B * T) // blk
    out = pl.pallas_call(
        _ln_kernel,
        grid=(nblk,),
        in_specs=[
            pl.BlockSpec((blk, H), lambda i: (i, 0)),
            pl.BlockSpec((blk, H), lambda i: (i % (T // blk), 0)),
            pl.BlockSpec((1, 1, blk), lambda i: (i, 0, 0)),
            pl.BlockSpec((2, H), lambda i: (0, 0)),
            pl.BlockSpec((1, H), lambda i: (0, 0)),
            pl.BlockSpec((1, H), lambda i: (0, 0)),
        ],
        out_specs=pl.BlockSpec((blk, H), lambda i: (i, 0)),
        out_shape=jax.ShapeDtypeStruct((B * T, H), jnp.float32),
    )(scratch, pos_emb[:T], tt3d, type_emb, gamma2d, beta2d)
    return out


def kernel(input_ids, token_type_ids, word_emb, pos_emb, type_emb, ln_gamma,
           ln_beta):
    assert input_ids.shape == (B, T) and word_emb.shape[1] == H
    ids_flat = input_ids.reshape(-1).astype(jnp.int32)
    tt3d = token_type_ids.reshape(B * T // 128, 1, 128).astype(jnp.int32)
    out = _run(ids_flat, tt3d, word_emb, pos_emb, type_emb,
               ln_gamma.reshape(1, H), ln_beta.reshape(1, H))
    return out.reshape(B, T, H)
